# Initial kernel scaffold; baseline (speedup 1.0000x reference)
import jax
import jax.numpy as jnp
from jax.experimental import pallas as pl


def kernel(q_sub, q_rel, hidden, edges, n_node, old_nodes_new_idx, rela_embed, Ws, Wr, Wqr, bqr, w_alpha_w, w_alpha_b, W_h):
    raise NotImplementedError("write your pallas kernel here")


# speedup vs baseline: 4.1721x; 4.1721x over previous
import jax
import jax.numpy as jnp
from jax import lax
from jax.experimental import pallas as pl
from jax.experimental.pallas import tpu as pltpu, tpu_sc as plsc

N_NODE = 10000
E = 320000
D = 128
R_PAD = 10240
B_PAD = 10240
CHUNK = 64
NUM_CHUNKS = E // CHUNK
NW = 32
AGG = 10000
ROWS_PER_TILE = 632


def _mm_body(x_ref, w_ref, o_ref):
    o_ref[...] = jnp.dot(x_ref[...], w_ref[...],
                         preferred_element_type=jnp.float32)


def _rel_body(x_ref, wr_ref, wqr_ref, bqr_ref, ob_ref, oc_ref):
    x = x_ref[...]
    ob_ref[...] = jnp.dot(x, wr_ref[...], preferred_element_type=jnp.float32)
    oc_ref[...] = (jnp.dot(x, wqr_ref[...], preferred_element_type=jnp.float32)
                   + bqr_ref[...])


def _fin_body(p_ref, w_ref, o_ref):
    x = p_ref[0] + p_ref[1]
    o_ref[...] = jnp.dot(x, w_ref[...], preferred_element_type=jnp.float32)


def _qgather_body(tabc_hbm, qrel_hbm, out_hbm, qidx_v, rows_v, sem):
    c = lax.axis_index("c")
    s = lax.axis_index("s")
    wid = s * 2 + c
    base = wid * (B_PAD // NW)

    def blk(j, carry):
        off = base + j * CHUNK
        pltpu.sync_copy(qrel_hbm.at[pl.ds(off, CHUNK)], qidx_v)
        pltpu.async_copy(tabc_hbm.at[qidx_v], rows_v, sem).wait()
        pltpu.sync_copy(rows_v, out_hbm.at[pl.ds(off, CHUNK)])
        return carry

    lax.fori_loop(0, (B_PAD // NW) // CHUNK, blk, 0)


def _sc_body(edges_hbm, tabA_hbm, tabB_hbm, tabC2_hbm,
             hid_hbm, rela_hbm, wvec_hbm, zeros_hbm, out_hbm,
             edg_v, sub_v, rel_v, qr_v, obj_v,
             a_v, b_v, c_v, hs_v, hr_v, m_v,
             w_v, agg_sh, sem):
    c = lax.axis_index("c")
    s = lax.axis_index("s")
    wid = s * 2 + c

    pltpu.sync_copy(wvec_hbm, w_v)
    row0 = s * ROWS_PER_TILE

    @pl.when(s < 15)
    def _():
        pltpu.sync_copy(zeros_hbm.at[pl.ds(row0, 632)],
                        agg_sh.at[pl.ds(row0, 632)])

    @pl.when(s == 15)
    def _():
        pltpu.sync_copy(zeros_hbm.at[pl.ds(row0, 520)],
                        agg_sh.at[pl.ds(row0, 520)])

    plsc.subcore_barrier()

    wv = [w_v[pl.ds(16 * k, 16)] for k in range(8)]
    sbv = w_v[pl.ds(128, 16)]

    base_chunks = NUM_CHUNKS // NW
    extra = NUM_CHUNKS - base_chunks * NW
    nchunks = jnp.where(wid < extra, base_chunks + 1, base_chunks)

    def chunk_body(i, carry):
        chunk = wid + i * NW
        pltpu.sync_copy(edges_hbm.at[pl.ds(chunk * (CHUNK * 6), CHUNK * 6)],
                        edg_v)
        for g in range(CHUNK // 16):
            idx6 = (lax.iota(jnp.int32, 16) + g * 16) * 6
            r16 = plsc.load_gather(edg_v, [idx6])
            rel16 = plsc.load_gather(edg_v, [idx6 + 2])
            sub16 = plsc.load_gather(edg_v, [idx6 + 4])
            obj16 = jnp.minimum(plsc.load_gather(edg_v, [idx6 + 5]),
                                N_NODE - 1)
            sl = pl.ds(g * 16, 16)
            sub_v[sl] = sub16
            rel_v[sl] = rel16
            qr_v[sl] = r16
            obj_v[sl] = obj16
        d1 = pltpu.async_copy(tabA_hbm.at[sub_v], a_v, sem)
        d2 = pltpu.async_copy(tabB_hbm.at[rel_v], b_v, sem)
        d3 = pltpu.async_copy(tabC2_hbm.at[qr_v], c_v, sem)
        d4 = pltpu.async_copy(hid_hbm.at[sub_v], hs_v, sem)
        d5 = pltpu.async_copy(rela_hbm.at[rel_v], hr_v, sem)
        d1.wait(); d2.wait(); d3.wait(); d4.wait(); d5.wait()

        def edge_body(e, carry2):
            p = jnp.zeros((16,), jnp.float32)
            for k in range(8):
                ksl = pl.ds(16 * k, 16)
                t = a_v[e, ksl] + b_v[e, ksl] + c_v[e, ksl]
                p = p + jnp.maximum(t, 0.0) * wv[k]
            sval = jnp.sum(p)
            svec = jnp.full((16,), sval, jnp.float32) + sbv
            alpha = 1.0 / (1.0 + jnp.exp(-svec))
            for k in range(8):
                ksl = pl.ds(16 * k, 16)
                m_v[e, ksl] = alpha * (hs_v[e, ksl] + hr_v[e, ksl])
            return carry2

        lax.fori_loop(0, CHUNK, edge_body, 0)
        pltpu.sync_copy(m_v, agg_sh.at[obj_v], add=True)
        return carry

    lax.fori_loop(0, nchunks, chunk_body, 0)
    plsc.subcore_barrier()

    @pl.when(s < 15)
    def _():
        pltpu.sync_copy(agg_sh.at[pl.ds(row0, 632)],
                        out_hbm.at[pl.ds(c * AGG + row0, 632)])

    @pl.when(s == 15)
    def _():
        pltpu.sync_copy(agg_sh.at[pl.ds(row0, 520)],
                        out_hbm.at[pl.ds(c * AGG + row0, 520)])


def kernel(q_sub, q_rel, hidden, edges, n_node, old_nodes_new_idx,
           rela_embed, Ws, Wr, Wqr, bqr, w_alpha_w, w_alpha_b, W_h):
    n_emb = rela_embed.shape[0]
    f32 = jnp.float32
    hidden = hidden.astype(f32)
    rela_p = jnp.concatenate(
        [rela_embed.astype(f32),
         jnp.zeros((R_PAD - n_emb, D), f32)], axis=0)

    tabA = pl.pallas_call(
        _mm_body,
        grid=(25,),
        in_specs=[pl.BlockSpec((400, D), lambda i: (i, 0)),
                  pl.BlockSpec((D, D), lambda i: (0, 0))],
        out_specs=pl.BlockSpec((400, D), lambda i: (i, 0)),
        out_shape=jax.ShapeDtypeStruct((N_NODE, D), f32),
    )(hidden, Ws.astype(f32))

    tabB, tabC = pl.pallas_call(
        _rel_body,
        grid=(20,),
        in_specs=[pl.BlockSpec((512, D), lambda i: (i, 0)),
                  pl.BlockSpec((D, D), lambda i: (0, 0)),
                  pl.BlockSpec((D, D), lambda i: (0, 0)),
                  pl.BlockSpec((1, D), lambda i: (0, 0))],
        out_specs=[pl.BlockSpec((512, D), lambda i: (i, 0)),
                   pl.BlockSpec((512, D), lambda i: (i, 0))],
        out_shape=[jax.ShapeDtypeStruct((R_PAD, D), f32),
                   jax.ShapeDtypeStruct((R_PAD, D), f32)],
    )(rela_p, Wr.astype(f32), Wqr.astype(f32),
      bqr.astype(f32).reshape(1, D))

    mesh = plsc.VectorSubcoreMesh(core_axis_name="c", subcore_axis_name="s")
    sc_params = pltpu.CompilerParams(needs_layout_passes=False)

    q_rel_p = jnp.concatenate(
        [q_rel.astype(jnp.int32),
         jnp.zeros((B_PAD - q_rel.shape[0],), jnp.int32)])
    qgather = pl.kernel(
        _qgather_body,
        out_type=jax.ShapeDtypeStruct((B_PAD, D), f32),
        mesh=mesh,
        scratch_types=[
            pltpu.VMEM((CHUNK,), jnp.int32),
            pltpu.VMEM((CHUNK, D), f32),
            pltpu.SemaphoreType.DMA,
        ],
        compiler_params=sc_params,
    )
    tabC2 = qgather(tabC, q_rel_p)

    edges_flat = edges.astype(jnp.int32).reshape(-1)
    wvec = jnp.concatenate([w_alpha_w.astype(f32).reshape(-1),
                            jnp.broadcast_to(w_alpha_b.astype(f32), (1,))[0]
                            * jnp.ones((16,), f32)])
    zeros_init = jnp.zeros((AGG, D), f32)

    sc_call = pl.kernel(
        _sc_body,
        out_type=jax.ShapeDtypeStruct((2 * AGG, D), f32),
        mesh=mesh,
        scratch_types=[
            pltpu.VMEM((CHUNK * 6,), jnp.int32),
            pltpu.VMEM((CHUNK,), jnp.int32),
            pltpu.VMEM((CHUNK,), jnp.int32),
            pltpu.VMEM((CHUNK,), jnp.int32),
            pltpu.VMEM((CHUNK,), jnp.int32),
            pltpu.VMEM((CHUNK, D), f32),
            pltpu.VMEM((CHUNK, D), f32),
            pltpu.VMEM((CHUNK, D), f32),
            pltpu.VMEM((CHUNK, D), f32),
            pltpu.VMEM((CHUNK, D), f32),
            pltpu.VMEM((CHUNK, D), f32),
            pltpu.VMEM((144,), f32),
            pltpu.VMEM_SHARED((AGG, D), f32),
            pltpu.SemaphoreType.DMA,
        ],
        compiler_params=sc_params,
    )
    partial = sc_call(edges_flat, tabA, tabB, tabC2,
                      hidden, rela_p, wvec, zeros_init)

    part3 = partial.reshape(2, AGG, D)
    out = pl.pallas_call(
        _fin_body,
        grid=(25,),
        in_specs=[pl.BlockSpec((2, 400, D), lambda i: (0, i, 0)),
                  pl.BlockSpec((D, D), lambda i: (0, 0))],
        out_specs=pl.BlockSpec((400, D), lambda i: (i, 0)),
        out_shape=jax.ShapeDtypeStruct((N_NODE, D), f32),
    )(part3, W_h.astype(f32))
    return out


# speedup vs baseline: 5.8143x; 1.3936x over previous
import jax
import jax.numpy as jnp
from jax import lax
from jax.experimental import pallas as pl
from jax.experimental.pallas import tpu as pltpu, tpu_sc as plsc

N_NODE = 10000
E = 320000
D = 128
R_PAD = 10240
B_PAD = 10240
QCHUNK = 64
CHUNK = 32
NUM_CHUNKS = E // CHUNK
NW = 32
T_ITER = 314
CHUNKS_PAD = NW * T_ITER
AGG = 10000
ROWS_PER_TILE = 632


def _mm_body(x_ref, w_ref, o_ref):
    o_ref[...] = jnp.dot(x_ref[...], w_ref[...],
                         preferred_element_type=jnp.float32)


def _rel_body(x_ref, wr_ref, wqr_ref, bqr_ref, ob_ref, oc_ref):
    x = x_ref[...]
    ob_ref[...] = jnp.dot(x, wr_ref[...], preferred_element_type=jnp.float32)
    oc_ref[...] = (jnp.dot(x, wqr_ref[...], preferred_element_type=jnp.float32)
                   + bqr_ref[...])


def _fin_body(p_ref, w_ref, o_ref):
    x = p_ref[0] + p_ref[1]
    o_ref[...] = jnp.dot(x, w_ref[...], preferred_element_type=jnp.float32)


def _qgather_body(tabc_hbm, qrel_hbm, out_hbm, qidx_v, rows_v, sem):
    c = lax.axis_index("c")
    s = lax.axis_index("s")
    wid = s * 2 + c
    base = wid * (B_PAD // NW)

    def blk(j, carry):
        off = base + j * QCHUNK
        pltpu.sync_copy(qrel_hbm.at[pl.ds(off, QCHUNK)], qidx_v)
        pltpu.async_copy(tabc_hbm.at[qidx_v], rows_v, sem).wait()
        pltpu.sync_copy(rows_v, out_hbm.at[pl.ds(off, QCHUNK)])
        return carry

    lax.fori_loop(0, (B_PAD // NW) // QCHUNK, blk, 0)


def _sc_body(edges_hbm, tabA_hbm, tabB_hbm, tabC2_hbm,
             hid_hbm, rela_hbm, wvec_hbm, zeros_hbm, out_hbm,
             edg0, edg1, cols0, cols1, obj0, obj1,
             rows0, rows1, m_v, w_v, agg_sh,
             sem_e0, sem_e1, sem_g0, sem_g1):
    c = lax.axis_index("c")
    s = lax.axis_index("s")
    wid = s * 2 + c

    edg = (edg0, edg1)
    cols = (cols0, cols1)
    obj = (obj0, obj1)
    rows = (rows0, rows1)
    sem_e = (sem_e0, sem_e1)
    sem_g = (sem_g0, sem_g1)

    pltpu.sync_copy(wvec_hbm, w_v)
    row0 = s * ROWS_PER_TILE

    @pl.when(s < 15)
    def _():
        pltpu.sync_copy(zeros_hbm.at[pl.ds(row0, 632)],
                        agg_sh.at[pl.ds(row0, 632)])

    @pl.when(s == 15)
    def _():
        pltpu.sync_copy(zeros_hbm.at[pl.ds(row0, 520)],
                        agg_sh.at[pl.ds(row0, 520)])

    plsc.subcore_barrier()

    wv = [w_v[pl.ds(16 * k, 16)] for k in range(8)]
    sbv = w_v[pl.ds(128, 16)]

    def fire_edges(ii, b):
        chunk = wid + ii * NW
        pltpu.async_copy(edges_hbm.at[pl.ds(chunk * (CHUNK * 6), CHUNK * 6)],
                         edg[b], sem_e[b])

    def drain_edges(b):
        pltpu.make_async_copy(edges_hbm.at[pl.ds(0, CHUNK * 6)],
                              edg[b], sem_e[b]).wait()

    def prep_and_fire(b):
        for g in range(CHUNK // 16):
            idx6 = (lax.iota(jnp.int32, 16) + g * 16) * 6
            r16 = plsc.load_gather(edg[b], [idx6])
            rel16 = plsc.load_gather(edg[b], [idx6 + 2])
            sub16 = plsc.load_gather(edg[b], [idx6 + 4])
            obj16 = jnp.minimum(plsc.load_gather(edg[b], [idx6 + 5]),
                                N_NODE - 1)
            sl = pl.ds(g * 16, 16)
            cols[b][0, sl] = sub16
            cols[b][1, sl] = rel16
            cols[b][2, sl] = r16
            obj[b][sl] = obj16
        sub_i = cols[b].at[0]
        rel_i = cols[b].at[1]
        qr_i = cols[b].at[2]
        pltpu.async_copy(tabA_hbm.at[sub_i], rows[b].at[pl.ds(0, CHUNK)],
                         sem_g[b])
        pltpu.async_copy(tabB_hbm.at[rel_i], rows[b].at[pl.ds(CHUNK, CHUNK)],
                         sem_g[b])
        pltpu.async_copy(tabC2_hbm.at[qr_i],
                         rows[b].at[pl.ds(2 * CHUNK, CHUNK)], sem_g[b])
        pltpu.async_copy(hid_hbm.at[sub_i],
                         rows[b].at[pl.ds(3 * CHUNK, CHUNK)], sem_g[b])
        pltpu.async_copy(rela_hbm.at[rel_i],
                         rows[b].at[pl.ds(4 * CHUNK, CHUNK)], sem_g[b])

    def drain_gathers(b):
        pltpu.make_async_copy(zeros_hbm.at[pl.ds(0, 5 * CHUNK)],
                              rows[b], sem_g[b]).wait()

    def compute_and_scatter(ii, b):
        j = wid + ii * NW

        @pl.when(j < NUM_CHUNKS)
        def _():
            r_v = rows[b]

            def edge_body(e, carry2):
                p = jnp.zeros((16,), jnp.float32)
                for k in range(8):
                    ksl = pl.ds(16 * k, 16)
                    t = (r_v[e, ksl] + r_v[CHUNK + e, ksl]
                         + r_v[2 * CHUNK + e, ksl])
                    p = p + jnp.maximum(t, 0.0) * wv[k]
                sval = jnp.sum(p)
                svec = jnp.full((16,), sval, jnp.float32) + sbv
                alpha = 1.0 / (1.0 + jnp.exp(-svec))
                for k in range(8):
                    ksl = pl.ds(16 * k, 16)
                    m_v[e, ksl] = alpha * (r_v[3 * CHUNK + e, ksl]
                                           + r_v[4 * CHUNK + e, ksl])
                return carry2

            lax.fori_loop(0, CHUNK, edge_body, 0)
            pltpu.sync_copy(m_v, agg_sh.at[obj[b]], add=True)

    fire_edges(0, 0)
    drain_edges(0)
    prep_and_fire(0)
    fire_edges(1, 1)

    def ring_body(i, carry):
        ii0 = 2 * i
        @pl.when(ii0 + 2 < T_ITER)
        def _():
            fire_edges(ii0 + 2, 0)

        drain_edges(1)
        prep_and_fire(1)
        drain_gathers(0)
        compute_and_scatter(ii0, 0)

        @pl.when(ii0 + 3 < T_ITER)
        def _():
            fire_edges(ii0 + 3, 1)

        @pl.when(ii0 + 2 < T_ITER)
        def _():
            drain_edges(0)
            prep_and_fire(0)

        drain_gathers(1)
        compute_and_scatter(ii0 + 1, 1)
        return carry

    lax.fori_loop(0, T_ITER // 2, ring_body, 0)
    plsc.subcore_barrier()

    @pl.when(s < 15)
    def _():
        pltpu.sync_copy(agg_sh.at[pl.ds(row0, 632)],
                        out_hbm.at[pl.ds(c * AGG + row0, 632)])

    @pl.when(s == 15)
    def _():
        pltpu.sync_copy(agg_sh.at[pl.ds(row0, 520)],
                        out_hbm.at[pl.ds(c * AGG + row0, 520)])


def kernel(q_sub, q_rel, hidden, edges, n_node, old_nodes_new_idx,
           rela_embed, Ws, Wr, Wqr, bqr, w_alpha_w, w_alpha_b, W_h):
    n_emb = rela_embed.shape[0]
    f32 = jnp.float32
    hidden = hidden.astype(f32)
    rela_p = jnp.concatenate(
        [rela_embed.astype(f32),
         jnp.zeros((R_PAD - n_emb, D), f32)], axis=0)

    tabA = pl.pallas_call(
        _mm_body,
        grid=(25,),
        in_specs=[pl.BlockSpec((400, D), lambda i: (i, 0)),
                  pl.BlockSpec((D, D), lambda i: (0, 0))],
        out_specs=pl.BlockSpec((400, D), lambda i: (i, 0)),
        out_shape=jax.ShapeDtypeStruct((N_NODE, D), f32),
    )(hidden, Ws.astype(f32))

    tabB, tabC = pl.pallas_call(
        _rel_body,
        grid=(20,),
        in_specs=[pl.BlockSpec((512, D), lambda i: (i, 0)),
                  pl.BlockSpec((D, D), lambda i: (0, 0)),
                  pl.BlockSpec((D, D), lambda i: (0, 0)),
                  pl.BlockSpec((1, D), lambda i: (0, 0))],
        out_specs=[pl.BlockSpec((512, D), lambda i: (i, 0)),
                   pl.BlockSpec((512, D), lambda i: (i, 0))],
        out_shape=[jax.ShapeDtypeStruct((R_PAD, D), f32),
                   jax.ShapeDtypeStruct((R_PAD, D), f32)],
    )(rela_p, Wr.astype(f32), Wqr.astype(f32),
      bqr.astype(f32).reshape(1, D))

    mesh = plsc.VectorSubcoreMesh(core_axis_name="c", subcore_axis_name="s")
    sc_params = pltpu.CompilerParams(needs_layout_passes=False)

    q_rel_p = jnp.concatenate(
        [q_rel.astype(jnp.int32),
         jnp.zeros((B_PAD - q_rel.shape[0],), jnp.int32)])
    qgather = pl.kernel(
        _qgather_body,
        out_type=jax.ShapeDtypeStruct((B_PAD, D), f32),
        mesh=mesh,
        scratch_types=[
            pltpu.VMEM((QCHUNK,), jnp.int32),
            pltpu.VMEM((QCHUNK, D), f32),
            pltpu.SemaphoreType.DMA,
        ],
        compiler_params=sc_params,
    )
    tabC2 = qgather(tabC, q_rel_p)

    edges_flat = edges.astype(jnp.int32).reshape(-1)
    edges_flat = jnp.concatenate(
        [edges_flat,
         jnp.zeros((CHUNKS_PAD * CHUNK * 6 - E * 6,), jnp.int32)])
    wvec = jnp.concatenate([w_alpha_w.astype(f32).reshape(-1),
                            jnp.broadcast_to(w_alpha_b.astype(f32), (1,))[0]
                            * jnp.ones((16,), f32)])
    zeros_init = jnp.zeros((AGG, D), f32)

    sc_call = pl.kernel(
        _sc_body,
        out_type=jax.ShapeDtypeStruct((2 * AGG, D), f32),
        mesh=mesh,
        scratch_types=[
            pltpu.VMEM((CHUNK * 6,), jnp.int32),
            pltpu.VMEM((CHUNK * 6,), jnp.int32),
            pltpu.VMEM((3, CHUNK), jnp.int32),
            pltpu.VMEM((3, CHUNK), jnp.int32),
            pltpu.VMEM((CHUNK,), jnp.int32),
            pltpu.VMEM((CHUNK,), jnp.int32),
            pltpu.VMEM((5 * CHUNK, D), f32),
            pltpu.VMEM((5 * CHUNK, D), f32),
            pltpu.VMEM((CHUNK, D), f32),
            pltpu.VMEM((144,), f32),
            pltpu.VMEM_SHARED((AGG, D), f32),
            pltpu.SemaphoreType.DMA,
            pltpu.SemaphoreType.DMA,
            pltpu.SemaphoreType.DMA,
            pltpu.SemaphoreType.DMA,
        ],
        compiler_params=sc_params,
    )
    partial = sc_call(edges_flat, tabA, tabB, tabC2,
                      hidden, rela_p, wvec, zeros_init)

    part3 = partial.reshape(2, AGG, D)
    out = pl.pallas_call(
        _fin_body,
        grid=(25,),
        in_specs=[pl.BlockSpec((2, 400, D), lambda i: (0, i, 0)),
                  pl.BlockSpec((D, D), lambda i: (0, 0))],
        out_specs=pl.BlockSpec((400, D), lambda i: (i, 0)),
        out_shape=jax.ShapeDtypeStruct((N_NODE, D), f32),
    )(part3, W_h.astype(f32))
    return out


# speedup vs baseline: 6.3412x; 1.0906x over previous
import jax
import jax.numpy as jnp
from jax import lax
from jax.experimental import pallas as pl
from jax.experimental.pallas import tpu as pltpu, tpu_sc as plsc

N_NODE = 10000
E = 320000
D = 128
R_PAD = 10240
B_PAD = 10240
QCHUNK = 64
CHUNK = 32
NUM_CHUNKS = E // CHUNK
NW = 32
T_ITER = 314
CHUNKS_PAD = NW * T_ITER
AGG = 10000
ROWS_PER_TILE = 632


def _mm_body(x_ref, w_ref, o_ref):
    o_ref[...] = jnp.dot(x_ref[...], w_ref[...],
                         preferred_element_type=jnp.float32)


def _rel_body(x_ref, wr_ref, wqr_ref, bqr_ref, ob_ref, oc_ref):
    x = x_ref[...]
    ob_ref[...] = jnp.dot(x, wr_ref[...], preferred_element_type=jnp.float32)
    oc_ref[...] = (jnp.dot(x, wqr_ref[...], preferred_element_type=jnp.float32)
                   + bqr_ref[...])


def _fin_body(p_ref, w_ref, o_ref):
    x = p_ref[0] + p_ref[1]
    o_ref[...] = jnp.dot(x, w_ref[...], preferred_element_type=jnp.float32)


def _qgather_body(tabc_hbm, qrel_hbm, out_hbm, qidx_v, rows_v, sem):
    c = lax.axis_index("c")
    s = lax.axis_index("s")
    wid = s * 2 + c
    base = wid * (B_PAD // NW)

    def blk(j, carry):
        off = base + j * QCHUNK
        pltpu.sync_copy(qrel_hbm.at[pl.ds(off, QCHUNK)], qidx_v)
        pltpu.async_copy(tabc_hbm.at[qidx_v], rows_v, sem).wait()
        pltpu.sync_copy(rows_v, out_hbm.at[pl.ds(off, QCHUNK)])
        return carry

    lax.fori_loop(0, (B_PAD // NW) // QCHUNK, blk, 0)


def _sc_body(edges_hbm, tabA_hbm, tabB_hbm, tabC2_hbm,
             hid_hbm, rela_hbm, wvec_hbm, out_hbm,
             edg0, edg1, cols0, cols1, obj0, obj1,
             rows0, rows1, m_v, w_v, agg_sh,
             sem_e0, sem_e1, sem_g0, sem_g1):
    c = lax.axis_index("c")
    s = lax.axis_index("s")
    wid = s * 2 + c

    edg = (edg0, edg1)
    cols = (cols0, cols1)
    obj = (obj0, obj1)
    rows = (rows0, rows1)
    sem_e = (sem_e0, sem_e1)
    sem_g = (sem_g0, sem_g1)

    pltpu.sync_copy(wvec_hbm, w_v)
    row0 = s * ROWS_PER_TILE

    zf = jnp.zeros((16,), jnp.float32)

    def zero_body(e, cc):
        for k in range(8):
            m_v[e, pl.ds(16 * k, 16)] = zf
        return cc

    lax.fori_loop(0, CHUNK, zero_body, 0)

    @pl.when(s < 15)
    def _():
        for t in range(19):
            pltpu.sync_copy(m_v, agg_sh.at[pl.ds(row0 + 32 * t, 32)])
        pltpu.sync_copy(m_v.at[pl.ds(0, 24)],
                        agg_sh.at[pl.ds(row0 + 608, 24)])

    @pl.when(s == 15)
    def _():
        for t in range(16):
            pltpu.sync_copy(m_v, agg_sh.at[pl.ds(row0 + 32 * t, 32)])
        pltpu.sync_copy(m_v.at[pl.ds(0, 8)],
                        agg_sh.at[pl.ds(row0 + 512, 8)])

    plsc.subcore_barrier()

    wv = [w_v[pl.ds(16 * k, 16)] for k in range(8)]
    sbv = w_v[pl.ds(128, 16)]

    def fire_edges(ii, b):
        chunk = wid + ii * NW
        pltpu.async_copy(edges_hbm.at[pl.ds(chunk * (CHUNK * 6), CHUNK * 6)],
                         edg[b], sem_e[b])

    def drain_edges(b):
        pltpu.make_async_copy(edges_hbm.at[pl.ds(0, CHUNK * 6)],
                              edg[b], sem_e[b]).wait()

    def prep_and_fire(b):
        for g in range(CHUNK // 16):
            idx6 = (lax.iota(jnp.int32, 16) + g * 16) * 6
            r16 = plsc.load_gather(edg[b], [idx6])
            rel16 = plsc.load_gather(edg[b], [idx6 + 2])
            sub16 = plsc.load_gather(edg[b], [idx6 + 4])
            obj16 = jnp.minimum(plsc.load_gather(edg[b], [idx6 + 5]),
                                N_NODE - 1)
            sl = pl.ds(g * 16, 16)
            cols[b][0, sl] = sub16
            cols[b][1, sl] = rel16
            cols[b][2, sl] = r16
            obj[b][sl] = obj16
        sub_i = cols[b].at[0]
        rel_i = cols[b].at[1]
        qr_i = cols[b].at[2]
        pltpu.async_copy(tabA_hbm.at[sub_i], rows[b].at[pl.ds(0, CHUNK)],
                         sem_g[b])
        pltpu.async_copy(tabB_hbm.at[rel_i], rows[b].at[pl.ds(CHUNK, CHUNK)],
                         sem_g[b])
        pltpu.async_copy(tabC2_hbm.at[qr_i],
                         rows[b].at[pl.ds(2 * CHUNK, CHUNK)], sem_g[b])
        pltpu.async_copy(hid_hbm.at[sub_i],
                         rows[b].at[pl.ds(3 * CHUNK, CHUNK)], sem_g[b])
        pltpu.async_copy(rela_hbm.at[rel_i],
                         rows[b].at[pl.ds(4 * CHUNK, CHUNK)], sem_g[b])

    def drain_gathers(b):
        pltpu.make_async_copy(tabA_hbm.at[pl.ds(0, 5 * CHUNK)],
                              rows[b], sem_g[b]).wait()

    def compute_and_scatter(b):
        r_v = rows[b]

        def edge_body(e, carry2):
            p = jnp.zeros((16,), jnp.float32)
            for k in range(8):
                ksl = pl.ds(16 * k, 16)
                t = (r_v[e, ksl] + r_v[CHUNK + e, ksl]
                     + r_v[2 * CHUNK + e, ksl])
                p = p + jnp.maximum(t, 0.0) * wv[k]
            sval = jnp.sum(p)
            svec = jnp.full((16,), sval, jnp.float32) + sbv
            alpha = 1.0 / (1.0 + jnp.exp(-svec))
            for k in range(8):
                ksl = pl.ds(16 * k, 16)
                m_v[e, ksl] = alpha * (r_v[3 * CHUNK + e, ksl]
                                       + r_v[4 * CHUNK + e, ksl])
            return carry2

        lax.fori_loop(0, CHUNK, edge_body, 0)
        pltpu.sync_copy(m_v, agg_sh.at[obj[b]], add=True)

    def valid(ii):
        return wid + ii * NW < NUM_CHUNKS

    fire_edges(0, 0)
    drain_edges(0)
    prep_and_fire(0)

    @pl.when(valid(1))
    def _():
        fire_edges(1, 1)

    def step(ii, b, nb):
        @pl.when(valid(ii + 2))
        def _():
            fire_edges(ii + 2, b)

        @pl.when(valid(ii + 1))
        def _():
            drain_edges(nb)
            prep_and_fire(nb)

        @pl.when(valid(ii))
        def _():
            drain_gathers(b)
            compute_and_scatter(b)

    def ring_body(i, carry):
        ii0 = 2 * i
        step(ii0, 0, 1)
        step(ii0 + 1, 1, 0)
        return carry

    lax.fori_loop(0, T_ITER // 2, ring_body, 0)
    plsc.subcore_barrier()

    @pl.when(s < 15)
    def _():
        pltpu.sync_copy(agg_sh.at[pl.ds(row0, 632)],
                        out_hbm.at[pl.ds(c * AGG + row0, 632)])

    @pl.when(s == 15)
    def _():
        pltpu.sync_copy(agg_sh.at[pl.ds(row0, 520)],
                        out_hbm.at[pl.ds(c * AGG + row0, 520)])


def kernel(q_sub, q_rel, hidden, edges, n_node, old_nodes_new_idx,
           rela_embed, Ws, Wr, Wqr, bqr, w_alpha_w, w_alpha_b, W_h):
    n_emb = rela_embed.shape[0]
    f32 = jnp.float32
    hidden = hidden.astype(f32)
    rela_p = jnp.concatenate(
        [rela_embed.astype(f32),
         jnp.zeros((R_PAD - n_emb, D), f32)], axis=0)

    tabA = pl.pallas_call(
        _mm_body,
        grid=(25,),
        in_specs=[pl.BlockSpec((400, D), lambda i: (i, 0)),
                  pl.BlockSpec((D, D), lambda i: (0, 0))],
        out_specs=pl.BlockSpec((400, D), lambda i: (i, 0)),
        out_shape=jax.ShapeDtypeStruct((N_NODE, D), f32),
    )(hidden, Ws.astype(f32))

    tabB, tabC = pl.pallas_call(
        _rel_body,
        grid=(20,),
        in_specs=[pl.BlockSpec((512, D), lambda i: (i, 0)),
                  pl.BlockSpec((D, D), lambda i: (0, 0)),
                  pl.BlockSpec((D, D), lambda i: (0, 0)),
                  pl.BlockSpec((1, D), lambda i: (0, 0))],
        out_specs=[pl.BlockSpec((512, D), lambda i: (i, 0)),
                   pl.BlockSpec((512, D), lambda i: (i, 0))],
        out_shape=[jax.ShapeDtypeStruct((R_PAD, D), f32),
                   jax.ShapeDtypeStruct((R_PAD, D), f32)],
    )(rela_p, Wr.astype(f32), Wqr.astype(f32),
      bqr.astype(f32).reshape(1, D))

    mesh = plsc.VectorSubcoreMesh(core_axis_name="c", subcore_axis_name="s")
    sc_params = pltpu.CompilerParams(needs_layout_passes=False)

    q_rel_p = jnp.concatenate(
        [q_rel.astype(jnp.int32),
         jnp.zeros((B_PAD - q_rel.shape[0],), jnp.int32)])
    qgather = pl.kernel(
        _qgather_body,
        out_type=jax.ShapeDtypeStruct((B_PAD, D), f32),
        mesh=mesh,
        scratch_types=[
            pltpu.VMEM((QCHUNK,), jnp.int32),
            pltpu.VMEM((QCHUNK, D), f32),
            pltpu.SemaphoreType.DMA,
        ],
        compiler_params=sc_params,
    )
    tabC2 = qgather(tabC, q_rel_p)

    edges_flat = edges.astype(jnp.int32).reshape(-1)
    wvec = jnp.concatenate([w_alpha_w.astype(f32).reshape(-1),
                            jnp.broadcast_to(w_alpha_b.astype(f32), (1,))[0]
                            * jnp.ones((16,), f32)])

    sc_call = pl.kernel(
        _sc_body,
        out_type=jax.ShapeDtypeStruct((2 * AGG, D), f32),
        mesh=mesh,
        scratch_types=[
            pltpu.VMEM((CHUNK * 6,), jnp.int32),
            pltpu.VMEM((CHUNK * 6,), jnp.int32),
            pltpu.VMEM((3, CHUNK), jnp.int32),
            pltpu.VMEM((3, CHUNK), jnp.int32),
            pltpu.VMEM((CHUNK,), jnp.int32),
            pltpu.VMEM((CHUNK,), jnp.int32),
            pltpu.VMEM((5 * CHUNK, D), f32),
            pltpu.VMEM((5 * CHUNK, D), f32),
            pltpu.VMEM((CHUNK, D), f32),
            pltpu.VMEM((144,), f32),
            pltpu.VMEM_SHARED((AGG, D), f32),
            pltpu.SemaphoreType.DMA,
            pltpu.SemaphoreType.DMA,
            pltpu.SemaphoreType.DMA,
            pltpu.SemaphoreType.DMA,
        ],
        compiler_params=sc_params,
    )
    partial = sc_call(edges_flat, tabA, tabB, tabC2,
                      hidden, rela_p, wvec)

    part3 = partial.reshape(2, AGG, D)
    out = pl.pallas_call(
        _fin_body,
        grid=(25,),
        in_specs=[pl.BlockSpec((2, 400, D), lambda i: (0, i, 0)),
                  pl.BlockSpec((D, D), lambda i: (0, 0))],
        out_specs=pl.BlockSpec((400, D), lambda i: (i, 0)),
        out_shape=jax.ShapeDtypeStruct((N_NODE, D), f32),
    )(part3, W_h.astype(f32))
    return out
